# baseline (speedup 1.0000x reference)
import jax
import jax.numpy as jnp
from jax.experimental import pallas as pl
from jax.experimental.pallas import tpu as pltpu

B = 128
V = 100000
CHUNK = 4096
NCHUNK = (V + CHUNK - 1) // CHUNK
LANES = 128
FOLDS = CHUNK // LANES

_NEG_INF = float("-inf")


def _margin_kernel(label_ref, logits_ref, out_ref, accmax_ref, acclab_ref):
    i = pl.program_id(0)
    x = logits_ref[...]
    lane = jax.lax.broadcasted_iota(jnp.int32, (B, LANES), 1)
    lab_rel = label_ref[...].reshape(B, 1) - i * CHUNK

    @pl.when(i == 0)
    def _init():
        accmax_ref[...] = jnp.full((B, LANES), _NEG_INF, jnp.float32)
        acclab_ref[...] = jnp.full((B, LANES), _NEG_INF, jnp.float32)

    def folds(extra_kill):
        acc_m = accmax_ref[...]
        acc_l = acclab_ref[...]
        for k in range(FOLDS):
            xs = x[:, k * LANES:(k + 1) * LANES]
            is_lab = lane == (lab_rel - k * LANES)
            if extra_kill is not None:
                kill = is_lab | (lane >= (extra_kill - k * LANES))
                labv = jnp.where(is_lab & (lane < (extra_kill - k * LANES)),
                                 xs, _NEG_INF)
            else:
                kill = is_lab
                labv = jnp.where(is_lab, xs, _NEG_INF)
            acc_m = jnp.maximum(acc_m, jnp.where(kill, _NEG_INF, xs))
            acc_l = jnp.maximum(acc_l, labv)
        return acc_m, acc_l

    @pl.when(i < NCHUNK - 1)
    def _body():
        acc_m, acc_l = folds(None)
        accmax_ref[...] = acc_m
        acclab_ref[...] = acc_l

    @pl.when(i == NCHUNK - 1)
    def _last():
        acc_m, acc_l = folds(V - (NCHUNK - 1) * CHUNK)
        max_other = jnp.max(acc_m, axis=1)
        logit_label = jnp.max(acc_l, axis=1)
        diff = logit_label - max_other
        out_ref[...] = -jnp.maximum(diff, 0.0)


@jax.jit
def kernel(logits, label):
    return pl.pallas_call(
        _margin_kernel,
        grid=(NCHUNK,),
        in_specs=[
            pl.BlockSpec((B,), lambda i: (0,)),
            pl.BlockSpec((B, CHUNK), lambda i: (0, i)),
        ],
        out_specs=pl.BlockSpec((B,), lambda i: (0,)),
        out_shape=jax.ShapeDtypeStruct((B,), jnp.float32),
        scratch_shapes=[
            pltpu.VMEM((B, LANES), jnp.float32),
            pltpu.VMEM((B, LANES), jnp.float32),
        ],
        compiler_params=pltpu.CompilerParams(
            dimension_semantics=("arbitrary",),
        ),
    )(label, logits)
